# baseline (speedup 1.0000x reference)
import functools
import math

import jax
import jax.numpy as jnp
from jax import lax
from jax.experimental import pallas as pl
from jax.experimental.pallas import tpu as pltpu

_D = 1024
_E = 16
_H = 4096
_T = 128
_CAP = float(_T)
_EPS = 1e-6
_DH = _D // 2


def _ffn_body(x0_ref, x1_ref, wg0_ref, wg1_ref, bg_ref, w1_ref, b1_ref,
              w2_ref, b2_ref, out_ref, gate_ref, lg_ref, hacc_ref):
    e = pl.program_id(0)
    jd = pl.program_id(1)

    @pl.when((e == 0) & (jd == 0))
    def _logits_half0():
        lg_ref[...] = jnp.dot(x0_ref[...], wg0_ref[0],
                              preferred_element_type=jnp.float32)

    @pl.when((e == 0) & (jd == 1))
    def _gate():
        logits = lg_ref[...] + jnp.dot(x1_ref[...], wg1_ref[0],
                                       preferred_element_type=jnp.float32)
        logits = logits + bg_ref[...]
        m = jnp.max(logits, axis=1, keepdims=True)
        ex = jnp.exp(logits - m)
        p = ex / jnp.sum(ex, axis=1, keepdims=True)
        iota = lax.broadcasted_iota(jnp.int32, (_T, _E), 1)
        pm = jnp.max(p, axis=1, keepdims=True)
        first = jnp.min(jnp.where(p >= pm, iota, _E), axis=1, keepdims=True)
        masked = jnp.where(iota == first, p, 0.0)
        denom = jnp.sum(masked, axis=0, keepdims=True) + _EPS
        gate_ref[...] = masked / denom * _CAP

    @pl.when(jd == 0)
    def _h_half0():
        hacc_ref[...] = jnp.dot(x0_ref[...], w1_ref[0],
                                preferred_element_type=jnp.float32)

    @pl.when(jd == 1)
    def _h_half1_and_combine():
        h = hacc_ref[...] + jnp.dot(x1_ref[...], w1_ref[0],
                                    preferred_element_type=jnp.float32)
        h = h + b1_ref[0]
        h = 0.5 * h * (1.0 + lax.erf(h * (1.0 / math.sqrt(2.0))))
        iota = lax.broadcasted_iota(jnp.int32, (_T, _E), 1)
        g = jnp.sum(jnp.where(iota == e, gate_ref[...], 0.0),
                    axis=1, keepdims=True)
        contrib = jnp.dot(g * h, w2_ref[0],
                          preferred_element_type=jnp.float32) + g * b2_ref[0]

        @pl.when(e == 0)
        def _():
            out_ref[...] = contrib

        @pl.when(e > 0)
        def _():
            out_ref[...] += contrib


def kernel(x, Wg, bg, W1, b1, W2, b2):
    out = pl.pallas_call(
        _ffn_body,
        grid=(_E, 2),
        in_specs=[
            pl.BlockSpec((_T, _DH), lambda e, j: (0, 0)),
            pl.BlockSpec((_T, _DH), lambda e, j: (0, 1)),
            pl.BlockSpec((1, _DH, _E), lambda e, j: (0, 0, 0)),
            pl.BlockSpec((1, _DH, _E), lambda e, j: (1, 0, 0)),
            pl.BlockSpec((1, _E), lambda e, j: (0, 0)),
            pl.BlockSpec((1, _DH, _H), lambda e, j: (e, j, 0)),
            pl.BlockSpec((1, 1, _H), lambda e, j: (e, 0, 0)),
            pl.BlockSpec((1, _H, _D), lambda e, j: (e, 0, 0)),
            pl.BlockSpec((1, 1, _D), lambda e, j: (e, 0, 0)),
        ],
        out_specs=pl.BlockSpec((_T, _D), lambda e, j: (0, 0)),
        out_shape=jax.ShapeDtypeStruct((_T, _D), jnp.float32),
        scratch_shapes=[
            pltpu.VMEM((_T, _E), jnp.float32),
            pltpu.VMEM((_T, _E), jnp.float32),
            pltpu.VMEM((_T, _H), jnp.float32),
        ],
        compiler_params=pltpu.CompilerParams(
            dimension_semantics=("arbitrary", "arbitrary"),
        ),
    )(x, x, Wg.reshape(2, _DH, _E), Wg.reshape(2, _DH, _E),
      bg.reshape(1, _E), W1, b1.reshape(_E, 1, _H), W2,
      b2.reshape(_E, 1, _D))
    return out
